# baseline (speedup 1.0000x reference)
import functools

import jax
import jax.numpy as jnp
from jax import lax
from jax.experimental import pallas as pl
from jax.experimental.pallas import tpu as pltpu
from jax.experimental.pallas import tpu_sc as plsc

N_CLASSES = 1000
BATCH = 16384

NC = 2
NS = 16
L = 16
C_SPLIT = 496
C_MAX = N_CLASSES - C_SPLIT
COLS_PER_W = BATCH // NS
C_BLK = 128
N_BLKS = COLS_PER_W // C_BLK

_mesh = plsc.VectorSubcoreMesh(core_axis_name="c", subcore_axis_name="s")


@functools.partial(
    pl.kernel,
    out_type=jax.ShapeDtypeStruct((N_CLASSES, BATCH), jnp.float32),
    mesh=_mesh,
    scratch_types=[
        pltpu.VMEM((COLS_PER_W,), jnp.int32),
        pltpu.VMEM((C_MAX, C_BLK), jnp.float32),
        pltpu.VMEM((C_MAX, C_BLK), jnp.float32),
        pltpu.SemaphoreType.DMA,
        pltpu.SemaphoreType.DMA,
    ],
    compiler_params=pltpu.CompilerParams(needs_layout_passes=False),
)
def _one_hot_t_sc(idx_hbm, z_hbm, out_hbm, idx_v, buf0, buf1, sem0, sem1):
    core = lax.axis_index("c")
    sub = lax.axis_index("s")
    col0 = sub * COLS_PER_W
    base = core * C_SPLIT

    pltpu.sync_copy(idx_hbm.at[pl.ds(col0, COLS_PER_W)], idx_v)
    pltpu.sync_copy(z_hbm, buf0)
    pltpu.sync_copy(z_hbm, buf1)

    zeros16 = jnp.zeros((L,), jnp.float32)
    ones16 = jnp.ones((L,), jnp.float32)
    lane = lax.iota(jnp.int32, L)
    nrows = C_SPLIT + core * (C_MAX - C_SPLIT)

    def _flip(buf, blk, vals):
        for g in range(C_BLK // L):
            idxv = idx_v[pl.ds(blk * C_BLK + g * L, L)]
            rel = idxv - base
            mask = (rel >= 0) & (rel < nrows)
            plsc.store_scatter(buf, (rel, lane + (g * L)), vals, mask=mask)

    def _dma_start(buf, blk, sem):
        cs = pl.ds(col0 + blk * C_BLK, C_BLK)

        @pl.when(core == 0)
        def _():
            pltpu.async_copy(
                buf.at[pl.ds(0, C_SPLIT)],
                out_hbm.at[pl.ds(0, C_SPLIT), cs], sem)

        @pl.when(core == 1)
        def _():
            pltpu.async_copy(buf, out_hbm.at[pl.ds(C_SPLIT, C_MAX), cs], sem)

    def _dma_wait(buf, sem):
        @pl.when(core == 0)
        def _():
            pltpu.make_async_copy(
                buf.at[pl.ds(0, C_SPLIT)],
                out_hbm.at[pl.ds(0, C_SPLIT), pl.ds(0, C_BLK)], sem).wait()

        @pl.when(core == 1)
        def _():
            pltpu.make_async_copy(
                buf, out_hbm.at[pl.ds(C_SPLIT, C_MAX), pl.ds(0, C_BLK)],
                sem).wait()

    bufs = (buf0, buf1)
    sems = (sem0, sem1)
    for c in range(N_BLKS):
        b = c % 2
        buf, sem = bufs[b], sems[b]
        if c >= 2:
            _dma_wait(buf, sem)
            _flip(buf, c - 2, zeros16)
        _flip(buf, c, ones16)
        _dma_start(buf, c, sem)
    _dma_wait(buf0, sem0)
    _dma_wait(buf1, sem1)


def kernel(inputs):
    idx = inputs.astype(jnp.int32)
    zblk = jnp.zeros((C_MAX, C_BLK), jnp.float32)
    out_t = _one_hot_t_sc(idx, zblk)
    return out_t.T
